# baseline (speedup 1.0000x reference)
import functools

import numpy as np
import jax
import jax.numpy as jnp
from jax import lax
from jax.experimental import pallas as pl
from jax.experimental.pallas import tpu as pltpu
from jax.experimental.pallas import tpu_sc as plsc

_B, _L, _D = 64, 2048, 64
_N = _B * _L

np.random.seed(0)
_PERM = np.argsort(np.random.permutation(np.arange(_N))).astype(np.int32)

_info = plsc.get_sparse_core_info()
_NC, _NS = _info.num_cores, _info.num_subcores
_NW = _NC * _NS
_RPW = _N // _NW
_GL = 128
_NG = _RPW // _GL
_CHUNK = 512
_GPC = _CHUNK // _GL
_NCHUNK = _RPW // _CHUNK

_IDX3 = _PERM.reshape(_NW, _NG, _GL)

_mesh = plsc.VectorSubcoreMesh(core_axis_name="c", subcore_axis_name="s")


@functools.partial(
    pl.kernel,
    mesh=_mesh,
    compiler_params=pltpu.CompilerParams(use_tc_tiling_on_sc=False),
    out_type=jax.ShapeDtypeStruct((_N, _D), jnp.float32),
    scratch_types=[
        pltpu.VMEM((_NG, _GL), jnp.int32),
        pltpu.VMEM((_CHUNK, _D), jnp.float32),
        pltpu.VMEM((_CHUNK, _D), jnp.float32),
        pltpu.SemaphoreType.DMA,
        pltpu.SemaphoreType.DMA,
        pltpu.SemaphoreType.DMA,
    ],
)
def _gather_rows(x_hbm, idx_hbm, out_hbm, idx_v, buf0, buf1, gsem, osem0, osem1):
    wid = lax.axis_index("s") * _NC + lax.axis_index("c")
    base = wid * _RPW
    pltpu.sync_copy(idx_hbm.at[wid], idx_v)
    bufs = (buf0, buf1)
    osems = (osem0, osem1)
    out_handles = [None, None]
    for c in range(_NCHUNK):
        b = c % 2
        if out_handles[b] is not None:
            out_handles[b].wait()
        handles = [
            pltpu.async_copy(
                x_hbm.at[idx_v.at[c * _GPC + j]],
                bufs[b].at[pl.ds(j * _GL, _GL)],
                gsem,
            )
            for j in range(_GPC)
        ]
        for h in handles:
            h.wait()
        out_handles[b] = pltpu.async_copy(
            bufs[b], out_hbm.at[pl.ds(base + c * _CHUNK, _CHUNK)], osems[b]
        )
    for h in out_handles:
        if h is not None:
            h.wait()


def kernel(x):
    xf = x.reshape(_N, _D)
    idx = jnp.asarray(_IDX3)
    out = _gather_rows(xf, idx)
    return out.reshape(_B, _L, _D)


# speedup vs baseline: 1.0300x; 1.0300x over previous
import functools

import numpy as np
import jax
import jax.numpy as jnp
from jax import lax
from jax.experimental import pallas as pl
from jax.experimental.pallas import tpu as pltpu
from jax.experimental.pallas import tpu_sc as plsc

_B, _L, _D = 64, 2048, 64
_N = _B * _L

np.random.seed(0)
_PERM = np.argsort(np.random.permutation(np.arange(_N))).astype(np.int32)

_info = plsc.get_sparse_core_info()
_NC, _NS = _info.num_cores, _info.num_subcores
_NW = _NC * _NS
_RPW = _N // _NW
_CHUNK = 512
_NCHUNK = _RPW // _CHUNK
_RING = 3

_IDX3 = _PERM.reshape(_NW, _NCHUNK, _CHUNK)

_mesh = plsc.VectorSubcoreMesh(core_axis_name="c", subcore_axis_name="s")


@functools.partial(
    pl.kernel,
    mesh=_mesh,
    compiler_params=pltpu.CompilerParams(use_tc_tiling_on_sc=False),
    out_type=jax.ShapeDtypeStruct((_N, _D), jnp.float32),
    scratch_types=[
        pltpu.VMEM((_NCHUNK, _CHUNK), jnp.int32),
    ]
    + [pltpu.VMEM((_CHUNK, _D), jnp.float32) for _ in range(_RING)]
    + [pltpu.SemaphoreType.DMA for _ in range(2 * _RING)],
)
def _gather_rows(x_hbm, idx_hbm, out_hbm, idx_v, *rest):
    bufs = rest[:_RING]
    gsems = rest[_RING : 2 * _RING]
    osems = rest[2 * _RING :]
    wid = lax.axis_index("s") * _NC + lax.axis_index("c")
    base = wid * _RPW
    pltpu.sync_copy(idx_hbm.at[wid], idx_v)
    gh = [
        pltpu.async_copy(x_hbm.at[idx_v.at[r]], bufs[r], gsems[r])
        for r in range(_RING)
    ]
    oh = [None] * _RING
    for c in range(_NCHUNK):
        s = c % _RING
        gh[s].wait()
        oh[s] = pltpu.async_copy(
            bufs[s], out_hbm.at[pl.ds(base + c * _CHUNK, _CHUNK)], osems[s]
        )
        if c + _RING < _NCHUNK:
            oh[s].wait()
            gh[s] = pltpu.async_copy(
                x_hbm.at[idx_v.at[c + _RING]], bufs[s], gsems[s]
            )
    for c in range(_NCHUNK - _RING, _NCHUNK):
        oh[c % _RING].wait()


def kernel(x):
    xf = x.reshape(_N, _D)
    idx = jnp.asarray(_IDX3)
    out = _gather_rows(xf, idx)
    return out.reshape(_B, _L, _D)


# speedup vs baseline: 1.1871x; 1.1525x over previous
import functools

import numpy as np
import jax
import jax.experimental.layout as jxl
import jax.numpy as jnp
from jax import lax
from jax.experimental import pallas as pl
from jax.experimental.pallas import tpu as pltpu
from jax.experimental.pallas import tpu_sc as plsc

_B, _L, _D = 64, 2048, 64
_N = _B * _L

np.random.seed(0)
_PERM = np.argsort(np.random.permutation(np.arange(_N))).astype(np.int32)

_info = plsc.get_sparse_core_info()
_NC, _NS = _info.num_cores, _info.num_subcores
_NW = _NC * _NS
_RPW = _N // _NW
_CHUNK = 512
_NCHUNK = _RPW // _CHUNK
_RING = 3

_IDX3 = _PERM.reshape(_NW, _NCHUNK, _CHUNK)

_mesh = plsc.VectorSubcoreMesh(core_axis_name="c", subcore_axis_name="s")


@functools.partial(
    pl.kernel,
    mesh=_mesh,
    compiler_params=pltpu.CompilerParams(use_tc_tiling_on_sc=False),
    out_type=jax.ShapeDtypeStruct((_N, _D), jnp.float32),
    scratch_types=[
        pltpu.VMEM((_NCHUNK, _CHUNK), jnp.int32),
    ]
    + [pltpu.VMEM((_CHUNK, _D), jnp.float32) for _ in range(_RING)]
    + [pltpu.SemaphoreType.DMA for _ in range(2 * _RING)],
)
def _gather_rows(x_hbm, idx_hbm, out_hbm, idx_v, *rest):
    bufs = rest[:_RING]
    gsems = rest[_RING : 2 * _RING]
    osems = rest[2 * _RING :]
    wid = lax.axis_index("s") * _NC + lax.axis_index("c")
    base = wid * _RPW
    pltpu.sync_copy(idx_hbm.at[wid], idx_v)
    gh = [
        pltpu.async_copy(x_hbm.at[idx_v.at[r]], bufs[r], gsems[r])
        for r in range(_RING)
    ]
    oh = [None] * _RING
    for c in range(_NCHUNK):
        s = c % _RING
        gh[s].wait()
        oh[s] = pltpu.async_copy(
            bufs[s], out_hbm.at[pl.ds(base + c * _CHUNK, _CHUNK)], osems[s]
        )
        if c + _RING < _NCHUNK:
            oh[s].wait()
            gh[s] = pltpu.async_copy(
                x_hbm.at[idx_v.at[c + _RING]], bufs[s], gsems[s]
            )
    for c in range(_NCHUNK - _RING, _NCHUNK):
        oh[c % _RING].wait()


def kernel(x):
    x = jxl.with_layout_constraint(x, jxl.Layout((0, 1, 2)))
    xf = x.reshape(_N, _D)
    idx = jnp.asarray(_IDX3)
    out = _gather_rows(xf, idx)
    y = out.reshape(_B, _L, _D)
    return jxl.with_layout_constraint(y, jxl.Layout((0, 1, 2)))
